# baseline (speedup 1.0000x reference)
import jax
import jax.numpy as jnp
from jax import lax
from jax.experimental import pallas as pl
from jax.experimental.pallas import tpu as pltpu
from jax.experimental.pallas import tpu_sc as plsc

_N = 5000
_D = 128
_A = 32
_R1 = 1000
_RB = 320
_NQ = 10240
_QTRASH = _NQ
_NH = 5120
_HTRASH = _N
_NEGBIAS = -1e30
_CPR = 160
_CBASE = _NQ + 32


def _mm_nt(a, b):
    return jax.lax.dot_general(
        a, b, (((1,), (1,)), ((), ())), preferred_element_type=jnp.float32)


def _prep_kernel(co_ref, no_ref, un_ref, h_ref, dv_ref,
                 wih_ref, whh_ref, bih_ref, bhh_ref,
                 wk_ref, bk_ref, wv_ref, bv_ref, wq_ref, bq_ref,
                 hnew0_ref, kn_ref, ku_ref, vh_ref, m1max_ref, kmax2_ref,
                 qmax2_ref):
    i = pl.program_id(0)
    co = co_ref[...]
    h = h_ref[...]
    gi = _mm_nt(co, wih_ref[...]) + bih_ref[...]
    gh = _mm_nt(h, whh_ref[...]) + bhh_ref[...]
    r = jax.nn.sigmoid(gi[:, :_D] + gh[:, :_D])
    z = jax.nn.sigmoid(gi[:, _D:2 * _D] + gh[:, _D:2 * _D])
    n = jnp.tanh(gi[:, 2 * _D:] + r * gh[:, 2 * _D:])
    h_m1 = (1.0 - z) * n + z * h

    m1 = dv_ref[:, 0:1] > 0
    hnew0_ref[...] = jnp.where(m1, h_m1, 0.0)
    blockmax = jnp.max(jnp.where(m1, h_m1, -jnp.inf), axis=0, keepdims=True)

    @pl.when(i == 0)
    def _():
        m1max_ref[...] = jnp.full_like(m1max_ref[...], -jnp.inf)
        kmax2_ref[...] = jnp.zeros_like(kmax2_ref[...])
        qmax2_ref[...] = jnp.zeros_like(qmax2_ref[...])

    m1max_ref[...] = jnp.maximum(m1max_ref[...], blockmax)

    no = no_ref[...]
    un = un_ref[...]
    bf16 = jnp.bfloat16
    kn16 = (_mm_nt(no, wk_ref[...]) + bk_ref[...]).astype(bf16)
    ku16 = (_mm_nt(un, wk_ref[...]) + bk_ref[...]).astype(bf16)
    kn_ref[...] = kn16
    ku_ref[...] = ku16
    vh_ref[...] = (_mm_nt(co, wv_ref[...]) + bv_ref[...]).astype(bf16)

    knf = kn16.astype(jnp.float32)
    kuf = ku16.astype(jnp.float32)
    k2 = jnp.maximum(jnp.max(jnp.sum(knf * knf, axis=1)),
                     jnp.max(jnp.sum(kuf * kuf, axis=1)))
    kmax2_ref[...] = jnp.maximum(kmax2_ref[...], k2)

    qnb = (_mm_nt(no, wq_ref[...]) + bq_ref[...]).astype(bf16).astype(jnp.float32)
    qub = (_mm_nt(un, wq_ref[...]) + bq_ref[...]).astype(bf16).astype(jnp.float32)
    q2 = jnp.maximum(jnp.max(jnp.sum(qnb * qnb, axis=1)),
                     jnp.max(jnp.sum(qub * qub, axis=1)))
    qmax2_ref[...] = jnp.maximum(qmax2_ref[...], q2)


def _rank_kernel(m23_hbm, rk_hbm, hsrc_hbm, cnt_hbm,
                 mask_v, rk_v, hsrc_v, bb_v, cnt_v, sem):
    wid = lax.axis_index("s") * 2 + lax.axis_index("c")

    @pl.when(wid == 0)
    def _():
        pltpu.sync_copy(m23_hbm, mask_v)
        io = lax.iota(jnp.int32, 16)
        bb_v[pl.ds(0, 16)] = jnp.zeros((16,), jnp.int32)

        def body(c, cnt):
            mv = mask_v[pl.ds(c * 16, 16)]
            m = mv != 0
            x = jnp.where(m, 1, 0)
            bb_v[pl.ds(16, 16)] = x
            x = x + bb_v[pl.ds(15, 16)]
            bb_v[pl.ds(16, 16)] = x
            x = x + bb_v[pl.ds(14, 16)]
            bb_v[pl.ds(16, 16)] = x
            x = x + bb_v[pl.ds(12, 16)]
            bb_v[pl.ds(16, 16)] = x
            x = x + bb_v[pl.ds(8, 16)]
            ranks = cnt + x - 1
            jv = c * 16 + io
            rk_v[pl.ds(c * 16, 16)] = jnp.where(m, ranks, _QTRASH)
            return cnt + x[15]

        cnt = lax.fori_loop(0, _NQ // 16, body, jnp.int32(0))

        def body2(c, carry):
            r = c * 16 + io
            m2c = mask_v[pl.ds(c * 16, 16)]
            off3 = jnp.minimum(c * 16 + _N, _NQ - 16)
            m3c = mask_v[pl.ds(off3, 16)]
            rk2 = rk_v[pl.ds(c * 16, 16)]
            rk3 = rk_v[pl.ds(off3, 16)]
            hsrc_v[pl.ds(c * 16, 16)] = jnp.where(
                r < _N,
                jnp.where(m3c != 0, rk3,
                          jnp.where(m2c != 0, rk2, _CBASE + r)),
                _CBASE + r)
            return carry

        lax.fori_loop(0, _NH // 16, body2, 0)
        cnt_v[...] = jnp.zeros((16,), jnp.int32) + cnt
        pltpu.sync_copy(rk_v, rk_hbm)
        pltpu.sync_copy(hsrc_v, hsrc_hbm)
        pltpu.sync_copy(cnt_v, cnt_hbm)


def _qscatter_kernel(qsrc_hbm, rk_hbm, qc_hbm, idx_v, rows_v, sem):
    w = lax.axis_index("s") * 2 + lax.axis_index("c")
    base = w * _RB
    pltpu.sync_copy(rk_hbm.at[pl.ds(base, _RB)], idx_v)
    pltpu.sync_copy(qsrc_hbm.at[pl.ds(base, _RB)], rows_v)

    def remap(c, carry):
        v = idx_v[pl.ds(c * 16, 16)]
        idx_v[pl.ds(c * 16, 16)] = jnp.where(v == _QTRASH, _QTRASH + w, v)
        return carry

    lax.fori_loop(0, _RB // 16, remap, 0)
    pltpu.async_copy(rows_v, qc_hbm.at[idx_v], sem).wait()


def _att_kernel(cnt_sref, qc_ref, kn_ref, ku_ref, vh_ref, bn_ref, bu_ref,
                m1max_ref, wq_ref, bq_ref,
                att_ref, m23max_ref, out_ref):
    i = pl.program_id(0)
    n23 = cnt_sref[0]
    inv = jnp.float32(1.0) / jnp.sqrt(jnp.float32(_A))

    @pl.when(i == 0)
    def _():
        m23max_ref[...] = jnp.full_like(m23max_ref[...], -jnp.inf)

    @pl.when(i * _RB < n23)
    def _():
        qsrc = qc_ref[...]
        qb = (_mm_nt(qsrc, wq_ref[...]) + bq_ref[...]).astype(jnp.bfloat16)
        p = (jnp.exp(_mm_nt(qb, kn_ref[...]) * inv + bn_ref[...]) +
             jnp.exp(_mm_nt(qb, ku_ref[...]) * inv + bu_ref[...]))
        l = jnp.sum(p, axis=1, keepdims=True)
        att = jnp.dot(p.astype(jnp.bfloat16), vh_ref[...],
                      preferred_element_type=jnp.float32) / l
        ath = jnp.tanh(att)
        att_ref[...] = ath

        rows = i * _RB + jax.lax.broadcasted_iota(jnp.int32, (_RB, 1), 0)
        bm = jnp.max(jnp.where(rows < n23, ath, -jnp.inf),
                     axis=0, keepdims=True)
        m23max_ref[...] = jnp.maximum(m23max_ref[...], bm)

    @pl.when(i == pl.num_programs(0) - 1)
    def _():
        out_m1 = m1max_ref[...]
        out_m23 = m23max_ref[...]
        has1 = jnp.isfinite(jnp.max(out_m1))
        has23 = jnp.isfinite(jnp.max(out_m23))
        out_ref[...] = jnp.where(
            ~has1, out_m23,
            jnp.where(~has23, out_m1, jnp.maximum(out_m1, out_m23)))


def _hmerge_kernel(cat_hbm, hsrc_hbm, hnew_hbm, sidx_v, crows_v, sem):
    w = lax.axis_index("s") * 2 + lax.axis_index("c")
    base = w * _CPR
    pltpu.sync_copy(hsrc_hbm.at[pl.ds(base, _CPR)], sidx_v)
    pltpu.async_copy(cat_hbm.at[sidx_v], crows_v, sem).wait()
    pltpu.sync_copy(crows_v, hnew_hbm.at[pl.ds(base, _CPR)])


def kernel(t, co_embeddings, divided, no_embeddings, unrelated_embeddings,
           hidden_state, w_ih, w_hh, b_ih, b_hh, wq, bq, wk, bk, wv, bv):
    f32 = jnp.float32
    bf16 = jnp.bfloat16
    i32 = jnp.int32
    h = (hidden_state if hidden_state is not None
         else jnp.zeros((_N, _D), dtype=co_embeddings.dtype))
    tpos = jnp.asarray(t) > 0
    dv_eff = jnp.where(tpos, divided, divided * jnp.array([1, 0, 0], divided.dtype))

    nblk1 = _N // _R1
    row_spec1 = lambda w: pl.BlockSpec((_R1, w), lambda i: (i, 0))
    full = lambda a: pl.BlockSpec(a.shape, lambda i, *_: tuple(0 for _ in a.shape))

    bih2 = b_ih.reshape(1, -1)
    bhh2 = b_hh.reshape(1, -1)
    bq2 = bq.reshape(1, -1)
    bk2 = bk.reshape(1, -1)
    bv2 = bv.reshape(1, -1)

    hnew0, k_n, k_u, v_h, m1max, kmax2, qmax2 = pl.pallas_call(
        _prep_kernel,
        grid=(nblk1,),
        in_specs=[row_spec1(_D), row_spec1(_D), row_spec1(_D), row_spec1(_D),
                  row_spec1(3),
                  full(w_ih), full(w_hh), full(bih2), full(bhh2),
                  full(wk), full(bk2), full(wv), full(bv2),
                  full(wq), full(bq2)],
        out_specs=[row_spec1(_D), row_spec1(_A), row_spec1(_A), row_spec1(_D),
                   pl.BlockSpec((1, _D), lambda i: (0, 0)),
                   pl.BlockSpec((1, _D), lambda i: (0, 0)),
                   pl.BlockSpec((1, _D), lambda i: (0, 0))],
        out_shape=[
            jax.ShapeDtypeStruct((_NH, _D), f32),
            jax.ShapeDtypeStruct((_N, _A), bf16),
            jax.ShapeDtypeStruct((_N, _A), bf16),
            jax.ShapeDtypeStruct((_N, _D), bf16),
            jax.ShapeDtypeStruct((1, _D), f32),
            jax.ShapeDtypeStruct((1, _D), f32),
            jax.ShapeDtypeStruct((1, _D), f32),
        ],
    )(co_embeddings, no_embeddings, unrelated_embeddings, h, dv_eff,
      w_ih, w_hh, bih2, bhh2, wk, bk2, wv, bv2, wq, bq2)

    mask2 = dv_eff[:, 1].astype(i32)
    mask3 = dv_eff[:, 2].astype(i32)
    m23pad = jnp.concatenate([mask2, mask3, jnp.zeros((_NQ - 2 * _N,), i32)])
    q_src = jnp.concatenate(
        [no_embeddings, unrelated_embeddings,
         jnp.zeros((_NQ - 2 * _N, _D), f32)], axis=0)
    m_g = jnp.sqrt(qmax2[0, 0] * kmax2[0, 0]) * (1.0 / jnp.sqrt(f32(_A))) + f32(1e-3)
    bias_n = jnp.where(mask2 > 0, f32(0), f32(_NEGBIAS)).reshape(1, _N) - m_g
    bias_u = jnp.where(mask3 > 0, f32(0), f32(_NEGBIAS)).reshape(1, _N) - m_g

    mesh = plsc.VectorSubcoreMesh(core_axis_name="c", subcore_axis_name="s")

    rk, hsrc, cnt = pl.kernel(
        _rank_kernel,
        out_type=[
            jax.ShapeDtypeStruct((_NQ,), i32),
            jax.ShapeDtypeStruct((_NH,), i32),
            jax.ShapeDtypeStruct((16,), i32),
        ],
        mesh=mesh,
        scratch_types=[
            pltpu.VMEM((_NQ,), i32),
            pltpu.VMEM((_NQ,), i32),
            pltpu.VMEM((_NH,), i32),
            pltpu.VMEM((32,), i32),
            pltpu.VMEM((16,), i32),
            pltpu.SemaphoreType.DMA,
        ],
    )(m23pad)

    q_c = pl.kernel(
        _qscatter_kernel,
        out_type=jax.ShapeDtypeStruct((_NQ + 32, _D), f32),
        mesh=mesh,
        scratch_types=[
            pltpu.VMEM((_RB,), i32),
            pltpu.VMEM((_RB, _D), f32),
            pltpu.SemaphoreType.DMA,
        ],
    )(q_src, rk)

    att_c, m23max, outv = pl.pallas_call(
        _att_kernel,
        grid_spec=pltpu.PrefetchScalarGridSpec(
            num_scalar_prefetch=1,
            grid=(_NQ // _RB,),
            in_specs=[pl.BlockSpec((_RB, _D), lambda i, *_: (i, 0)),
                      full(k_n), full(k_u), full(v_h),
                      full(bias_n), full(bias_u),
                      full(m1max), full(wq), full(bq2)],
            out_specs=[pl.BlockSpec((_RB, _D), lambda i, *_: (i, 0)),
                       pl.BlockSpec((1, _D), lambda i, *_: (0, 0)),
                       pl.BlockSpec((1, _D), lambda i, *_: (0, 0))],
        ),
        out_shape=[
            jax.ShapeDtypeStruct((_NQ + 32, _D), f32),
            jax.ShapeDtypeStruct((1, _D), f32),
            jax.ShapeDtypeStruct((1, _D), f32),
        ],
    )(cnt, q_c, k_n, k_u, v_h, bias_n, bias_u, m1max, wq, bq2)

    cat = jnp.concatenate([att_c, hnew0], axis=0)
    h_new_full = pl.kernel(
        _hmerge_kernel,
        out_type=jax.ShapeDtypeStruct((_NH, _D), f32),
        mesh=mesh,
        scratch_types=[
            pltpu.VMEM((_CPR,), i32),
            pltpu.VMEM((_CPR, _D), f32),
            pltpu.SemaphoreType.DMA,
        ],
    )(cat, hsrc)

    return (outv[0], h_new_full[:_N])


# speedup vs baseline: 1.0226x; 1.0226x over previous
import jax
import jax.numpy as jnp
from jax import lax
from jax.experimental import pallas as pl
from jax.experimental.pallas import tpu as pltpu
from jax.experimental.pallas import tpu_sc as plsc

_N = 5000
_D = 128
_A = 32
_R1 = 1000
_RB = 320
_NQ = 10240
_QTRASH = _NQ
_NH = 5120
_HTRASH = _N
_NEGBIAS = -1e30
_CPR = 160
_CBASE = _NQ + 32


def _mm_nt(a, b):
    return jax.lax.dot_general(
        a, b, (((1,), (1,)), ((), ())), preferred_element_type=jnp.float32)


def _prep_kernel(co_ref, no_ref, un_ref, h_ref, dv_ref,
                 wih_ref, whh_ref, bih_ref, bhh_ref,
                 wk_ref, bk_ref, wv_ref, bv_ref, wq_ref, bq_ref,
                 hnew0_ref, kn_ref, ku_ref, vh_ref, m1max_ref, kmax2_ref,
                 qmax2_ref):
    i = pl.program_id(0)
    co = co_ref[...]
    h = h_ref[...]
    gi = _mm_nt(co, wih_ref[...]) + bih_ref[...]
    gh = _mm_nt(h, whh_ref[...]) + bhh_ref[...]
    r = jax.nn.sigmoid(gi[:, :_D] + gh[:, :_D])
    z = jax.nn.sigmoid(gi[:, _D:2 * _D] + gh[:, _D:2 * _D])
    n = jnp.tanh(gi[:, 2 * _D:] + r * gh[:, 2 * _D:])
    h_m1 = (1.0 - z) * n + z * h

    m1 = dv_ref[:, 0:1] > 0
    hnew0_ref[...] = jnp.where(m1, h_m1, 0.0)
    blockmax = jnp.max(jnp.where(m1, h_m1, -jnp.inf), axis=0, keepdims=True)

    @pl.when(i == 0)
    def _():
        m1max_ref[...] = jnp.full_like(m1max_ref[...], -jnp.inf)
        kmax2_ref[...] = jnp.zeros_like(kmax2_ref[...])
        qmax2_ref[...] = jnp.zeros_like(qmax2_ref[...])

    m1max_ref[...] = jnp.maximum(m1max_ref[...], blockmax)

    no = no_ref[...]
    un = un_ref[...]
    bf16 = jnp.bfloat16
    kn16 = (_mm_nt(no, wk_ref[...]) + bk_ref[...]).astype(bf16)
    ku16 = (_mm_nt(un, wk_ref[...]) + bk_ref[...]).astype(bf16)
    kn_ref[...] = kn16
    ku_ref[...] = ku16
    vh_ref[...] = (_mm_nt(co, wv_ref[...]) + bv_ref[...]).astype(bf16)

    knf = kn16.astype(jnp.float32)
    kuf = ku16.astype(jnp.float32)
    k2 = jnp.maximum(jnp.max(jnp.sum(knf * knf, axis=1)),
                     jnp.max(jnp.sum(kuf * kuf, axis=1)))
    kmax2_ref[...] = jnp.maximum(kmax2_ref[...], k2)

    qnb = (_mm_nt(no, wq_ref[...]) + bq_ref[...]).astype(bf16).astype(jnp.float32)
    qub = (_mm_nt(un, wq_ref[...]) + bq_ref[...]).astype(bf16).astype(jnp.float32)
    q2 = jnp.maximum(jnp.max(jnp.sum(qnb * qnb, axis=1)),
                     jnp.max(jnp.sum(qub * qub, axis=1)))
    qmax2_ref[...] = jnp.maximum(qmax2_ref[...], q2)


def _rank_kernel(m23_hbm, rk_hbm, hsrc_hbm, cnt_hbm,
                 mask_v, rk_v, hsrc_v, bb_v, cnt_v, sem):
    wid = lax.axis_index("s") * 2 + lax.axis_index("c")

    @pl.when(wid == 0)
    def _():
        pltpu.sync_copy(m23_hbm, mask_v)
        io = lax.iota(jnp.int32, 16)
        bb_v[pl.ds(0, 16)] = jnp.zeros((16,), jnp.int32)

        def body(c, cnt):
            mv = mask_v[pl.ds(c * 16, 16)]
            m = mv != 0
            x = jnp.where(m, 1, 0)
            bb_v[pl.ds(16, 16)] = x
            x = x + bb_v[pl.ds(15, 16)]
            bb_v[pl.ds(16, 16)] = x
            x = x + bb_v[pl.ds(14, 16)]
            bb_v[pl.ds(16, 16)] = x
            x = x + bb_v[pl.ds(12, 16)]
            bb_v[pl.ds(16, 16)] = x
            x = x + bb_v[pl.ds(8, 16)]
            ranks = cnt + x - 1
            jv = c * 16 + io
            rk_v[pl.ds(c * 16, 16)] = jnp.where(m, ranks, _QTRASH)
            return cnt + x[15]

        cnt = lax.fori_loop(0, _NQ // 16, body, jnp.int32(0))

        def body2(c, carry):
            r = c * 16 + io
            m2c = mask_v[pl.ds(c * 16, 16)]
            off3 = jnp.minimum(c * 16 + _N, _NQ - 16)
            m3c = mask_v[pl.ds(off3, 16)]
            rk2 = rk_v[pl.ds(c * 16, 16)]
            rk3 = rk_v[pl.ds(off3, 16)]
            hsrc_v[pl.ds(c * 16, 16)] = jnp.where(
                r < _N,
                jnp.where(m3c != 0, rk3,
                          jnp.where(m2c != 0, rk2, _CBASE + r)),
                _CBASE + r)
            return carry

        lax.fori_loop(0, _NH // 16, body2, 0)
        cnt_v[...] = jnp.zeros((16,), jnp.int32) + cnt
        pltpu.sync_copy(rk_v, rk_hbm)
        pltpu.sync_copy(hsrc_v, hsrc_hbm)
        pltpu.sync_copy(cnt_v, cnt_hbm)


def _qscatter_kernel(no_hbm, un_hbm, rk_hbm, qc_hbm, idx_v, rows_v, sem):
    w = lax.axis_index("s") * 2 + lax.axis_index("c")
    base = w * _RB
    pltpu.sync_copy(rk_hbm.at[pl.ds(base, _RB)], idx_v)

    nsplit = _N // _RB
    @pl.when(w < nsplit)
    def _():
        pltpu.sync_copy(no_hbm.at[pl.ds(base, _RB)], rows_v)

    @pl.when(w == nsplit)
    def _():
        lo = _N - nsplit * _RB
        pltpu.sync_copy(no_hbm.at[pl.ds(nsplit * _RB, lo)],
                        rows_v.at[pl.ds(0, lo)])
        pltpu.sync_copy(un_hbm.at[pl.ds(0, _RB - lo)],
                        rows_v.at[pl.ds(lo, _RB - lo)])

    @pl.when((w > nsplit) & (base - _N + _RB <= _N))
    def _():
        pltpu.sync_copy(un_hbm.at[pl.ds(base - _N, _RB)], rows_v)

    @pl.when(base - _N + _RB > _N)
    def _():
        pltpu.sync_copy(un_hbm.at[pl.ds(base - _N, 2 * _N - base)],
                        rows_v.at[pl.ds(0, 2 * _N - base)])

    def remap(c, carry):
        v = idx_v[pl.ds(c * 16, 16)]
        idx_v[pl.ds(c * 16, 16)] = jnp.where(v == _QTRASH, _QTRASH + w, v)
        return carry

    lax.fori_loop(0, _RB // 16, remap, 0)
    pltpu.async_copy(rows_v, qc_hbm.at[idx_v], sem).wait()


def _att_kernel(cnt_sref, qc_ref, kn_ref, ku_ref, vh_ref, bn_ref, bu_ref,
                m1max_ref, wq_ref, bq_ref,
                att_ref, m23max_ref, out_ref):
    i = pl.program_id(0)
    n23 = cnt_sref[0]
    inv = jnp.float32(1.0) / jnp.sqrt(jnp.float32(_A))

    @pl.when(i == 0)
    def _():
        m23max_ref[...] = jnp.full_like(m23max_ref[...], -jnp.inf)

    @pl.when(i * _RB < n23)
    def _():
        qsrc = qc_ref[...]
        qb = (_mm_nt(qsrc, wq_ref[...]) + bq_ref[...]).astype(jnp.bfloat16)
        p = (jnp.exp(_mm_nt(qb, kn_ref[...]) * inv + bn_ref[...]) +
             jnp.exp(_mm_nt(qb, ku_ref[...]) * inv + bu_ref[...]))
        l = jnp.sum(p, axis=1, keepdims=True)
        att = jnp.dot(p.astype(jnp.bfloat16), vh_ref[...],
                      preferred_element_type=jnp.float32) / l
        ath = jnp.tanh(att)
        att_ref[...] = ath

        rows = i * _RB + jax.lax.broadcasted_iota(jnp.int32, (_RB, 1), 0)
        bm = jnp.max(jnp.where(rows < n23, ath, -jnp.inf),
                     axis=0, keepdims=True)
        m23max_ref[...] = jnp.maximum(m23max_ref[...], bm)

    @pl.when(i == pl.num_programs(0) - 1)
    def _():
        out_m1 = m1max_ref[...]
        out_m23 = m23max_ref[...]
        has1 = jnp.isfinite(jnp.max(out_m1))
        has23 = jnp.isfinite(jnp.max(out_m23))
        out_ref[...] = jnp.where(
            ~has1, out_m23,
            jnp.where(~has23, out_m1, jnp.maximum(out_m1, out_m23)))


def _hmerge_kernel(cat_hbm, hsrc_hbm, hnew_hbm, sidx_v, crows_v, sem):
    w = lax.axis_index("s") * 2 + lax.axis_index("c")
    base = w * _CPR
    pltpu.sync_copy(hsrc_hbm.at[pl.ds(base, _CPR)], sidx_v)
    pltpu.async_copy(cat_hbm.at[sidx_v], crows_v, sem).wait()

    @pl.when(base + _CPR <= _N)
    def _():
        pltpu.sync_copy(crows_v, hnew_hbm.at[pl.ds(base, _CPR)])

    @pl.when(base + _CPR > _N)
    def _():
        pltpu.sync_copy(crows_v.at[pl.ds(0, _N - (31 * _CPR))],
                        hnew_hbm.at[pl.ds(31 * _CPR, _N - (31 * _CPR))])


def kernel(t, co_embeddings, divided, no_embeddings, unrelated_embeddings,
           hidden_state, w_ih, w_hh, b_ih, b_hh, wq, bq, wk, bk, wv, bv):
    f32 = jnp.float32
    bf16 = jnp.bfloat16
    i32 = jnp.int32
    h = (hidden_state if hidden_state is not None
         else jnp.zeros((_N, _D), dtype=co_embeddings.dtype))
    tpos = jnp.asarray(t) > 0
    dv_eff = jnp.where(tpos, divided, divided * jnp.array([1, 0, 0], divided.dtype))

    nblk1 = _N // _R1
    row_spec1 = lambda w: pl.BlockSpec((_R1, w), lambda i: (i, 0))
    full = lambda a: pl.BlockSpec(a.shape, lambda i, *_: tuple(0 for _ in a.shape))

    bih2 = b_ih.reshape(1, -1)
    bhh2 = b_hh.reshape(1, -1)
    bq2 = bq.reshape(1, -1)
    bk2 = bk.reshape(1, -1)
    bv2 = bv.reshape(1, -1)

    hnew0, k_n, k_u, v_h, m1max, kmax2, qmax2 = pl.pallas_call(
        _prep_kernel,
        grid=(nblk1,),
        in_specs=[row_spec1(_D), row_spec1(_D), row_spec1(_D), row_spec1(_D),
                  row_spec1(3),
                  full(w_ih), full(w_hh), full(bih2), full(bhh2),
                  full(wk), full(bk2), full(wv), full(bv2),
                  full(wq), full(bq2)],
        out_specs=[row_spec1(_D), row_spec1(_A), row_spec1(_A), row_spec1(_D),
                   pl.BlockSpec((1, _D), lambda i: (0, 0)),
                   pl.BlockSpec((1, _D), lambda i: (0, 0)),
                   pl.BlockSpec((1, _D), lambda i: (0, 0))],
        out_shape=[
            jax.ShapeDtypeStruct((_NH, _D), f32),
            jax.ShapeDtypeStruct((_N, _A), bf16),
            jax.ShapeDtypeStruct((_N, _A), bf16),
            jax.ShapeDtypeStruct((_N, _D), bf16),
            jax.ShapeDtypeStruct((1, _D), f32),
            jax.ShapeDtypeStruct((1, _D), f32),
            jax.ShapeDtypeStruct((1, _D), f32),
        ],
    )(co_embeddings, no_embeddings, unrelated_embeddings, h, dv_eff,
      w_ih, w_hh, bih2, bhh2, wk, bk2, wv, bv2, wq, bq2)

    mask2 = dv_eff[:, 1].astype(i32)
    mask3 = dv_eff[:, 2].astype(i32)
    m23pad = jnp.concatenate([mask2, mask3, jnp.zeros((_NQ - 2 * _N,), i32)])
    m_g = jnp.sqrt(qmax2[0, 0] * kmax2[0, 0]) * (1.0 / jnp.sqrt(f32(_A))) + f32(1e-3)
    bias_n = jnp.where(mask2 > 0, f32(0), f32(_NEGBIAS)).reshape(1, _N) - m_g
    bias_u = jnp.where(mask3 > 0, f32(0), f32(_NEGBIAS)).reshape(1, _N) - m_g

    mesh = plsc.VectorSubcoreMesh(core_axis_name="c", subcore_axis_name="s")

    rk, hsrc, cnt = pl.kernel(
        _rank_kernel,
        out_type=[
            jax.ShapeDtypeStruct((_NQ,), i32),
            jax.ShapeDtypeStruct((_NH,), i32),
            jax.ShapeDtypeStruct((16,), i32),
        ],
        mesh=mesh,
        scratch_types=[
            pltpu.VMEM((_NQ,), i32),
            pltpu.VMEM((_NQ,), i32),
            pltpu.VMEM((_NH,), i32),
            pltpu.VMEM((32,), i32),
            pltpu.VMEM((16,), i32),
            pltpu.SemaphoreType.DMA,
        ],
    )(m23pad)

    q_c = pl.kernel(
        _qscatter_kernel,
        out_type=jax.ShapeDtypeStruct((_NQ + 32, _D), f32),
        mesh=mesh,
        scratch_types=[
            pltpu.VMEM((_RB,), i32),
            pltpu.VMEM((_RB, _D), f32),
            pltpu.SemaphoreType.DMA,
        ],
    )(no_embeddings, unrelated_embeddings, rk)

    att_c, m23max, outv = pl.pallas_call(
        _att_kernel,
        grid_spec=pltpu.PrefetchScalarGridSpec(
            num_scalar_prefetch=1,
            grid=(_NQ // _RB,),
            in_specs=[pl.BlockSpec((_RB, _D), lambda i, *_: (i, 0)),
                      full(k_n), full(k_u), full(v_h),
                      full(bias_n), full(bias_u),
                      full(m1max), full(wq), full(bq2)],
            out_specs=[pl.BlockSpec((_RB, _D), lambda i, *_: (i, 0)),
                       pl.BlockSpec((1, _D), lambda i, *_: (0, 0)),
                       pl.BlockSpec((1, _D), lambda i, *_: (0, 0))],
        ),
        out_shape=[
            jax.ShapeDtypeStruct((_NQ + 32, _D), f32),
            jax.ShapeDtypeStruct((1, _D), f32),
            jax.ShapeDtypeStruct((1, _D), f32),
        ],
    )(cnt, q_c, k_n, k_u, v_h, bias_n, bias_u, m1max, wq, bq2)

    cat = jnp.concatenate([att_c, hnew0], axis=0)
    h_new = pl.kernel(
        _hmerge_kernel,
        out_type=jax.ShapeDtypeStruct((_N, _D), f32),
        mesh=mesh,
        scratch_types=[
            pltpu.VMEM((_CPR,), i32),
            pltpu.VMEM((_CPR, _D), f32),
            pltpu.SemaphoreType.DMA,
        ],
    )(cat, hsrc)

    return (outv[0], h_new)


# speedup vs baseline: 1.0246x; 1.0019x over previous
import jax
import jax.numpy as jnp
from jax import lax
from jax.experimental import pallas as pl
from jax.experimental.pallas import tpu as pltpu
from jax.experimental.pallas import tpu_sc as plsc

_N = 5000
_D = 128
_A = 32
_R1 = 1000
_RB = 320
_NQ = 10240
_QTRASH = _NQ
_NH = 5120
_NEGBIAS = -1e30
_CPR = 160
_CBASE = _NQ + 32


def _mm_nt(a, b):
    return jax.lax.dot_general(
        a, b, (((1,), (1,)), ((), ())), preferred_element_type=jnp.float32)


def _prep_kernel(co_ref, no_ref, un_ref, h_ref, dv_ref,
                 wih_ref, whh_ref, bih_ref, bhh_ref,
                 wk_ref, bk_ref, wv_ref, bv_ref, wq_ref, bq_ref,
                 hnew0_ref, kn_ref, ku_ref, vh_ref, m1max_ref, kmax2_ref,
                 qmax2_ref):
    i = pl.program_id(0)
    co = co_ref[...]
    h = h_ref[...]
    gi = _mm_nt(co, wih_ref[...]) + bih_ref[...]
    gh = _mm_nt(h, whh_ref[...]) + bhh_ref[...]
    r = jax.nn.sigmoid(gi[:, :_D] + gh[:, :_D])
    z = jax.nn.sigmoid(gi[:, _D:2 * _D] + gh[:, _D:2 * _D])
    n = jnp.tanh(gi[:, 2 * _D:] + r * gh[:, 2 * _D:])
    h_m1 = (1.0 - z) * n + z * h

    m1 = dv_ref[:, 0:1] > 0
    hnew0_ref[...] = jnp.where(m1, h_m1, 0.0)
    blockmax = jnp.max(jnp.where(m1, h_m1, -jnp.inf), axis=0, keepdims=True)

    @pl.when(i == 0)
    def _():
        m1max_ref[...] = jnp.full_like(m1max_ref[...], -jnp.inf)
        kmax2_ref[...] = jnp.zeros_like(kmax2_ref[...])
        qmax2_ref[...] = jnp.zeros_like(qmax2_ref[...])

    m1max_ref[...] = jnp.maximum(m1max_ref[...], blockmax)

    no = no_ref[...]
    un = un_ref[...]
    bf16 = jnp.bfloat16
    kn16 = (_mm_nt(no, wk_ref[...]) + bk_ref[...]).astype(bf16)
    ku16 = (_mm_nt(un, wk_ref[...]) + bk_ref[...]).astype(bf16)
    kn_ref[...] = kn16
    ku_ref[...] = ku16
    vh_ref[...] = (_mm_nt(co, wv_ref[...]) + bv_ref[...]).astype(bf16)

    knf = kn16.astype(jnp.float32)
    kuf = ku16.astype(jnp.float32)
    k2 = jnp.maximum(jnp.max(jnp.sum(knf * knf, axis=1)),
                     jnp.max(jnp.sum(kuf * kuf, axis=1)))
    kmax2_ref[...] = jnp.maximum(kmax2_ref[...], k2)

    qnb = (_mm_nt(no, wq_ref[...]) + bq_ref[...]).astype(bf16).astype(jnp.float32)
    qub = (_mm_nt(un, wq_ref[...]) + bq_ref[...]).astype(bf16).astype(jnp.float32)
    q2 = jnp.maximum(jnp.max(jnp.sum(qnb * qnb, axis=1)),
                     jnp.max(jnp.sum(qub * qub, axis=1)))
    qmax2_ref[...] = jnp.maximum(qmax2_ref[...], q2)


def _rank_kernel(m23_hbm, rk_hbm, hsrc_hbm, cnt_hbm,
                 mask_v, rk_v, hsrc_v, bb_v, cnt_v, sem):
    wid = lax.axis_index("s") * 2 + lax.axis_index("c")

    @pl.when(wid == 0)
    def _():
        pltpu.sync_copy(m23_hbm, mask_v)
        io = lax.iota(jnp.int32, 16)
        bb_v[pl.ds(0, 16)] = jnp.zeros((16,), jnp.int32)

        def body(c, cnt):
            mv = mask_v[pl.ds(c * 16, 16)]
            m = mv != 0
            x = jnp.where(m, 1, 0)
            bb_v[pl.ds(16, 16)] = x
            x = x + bb_v[pl.ds(15, 16)]
            bb_v[pl.ds(16, 16)] = x
            x = x + bb_v[pl.ds(14, 16)]
            bb_v[pl.ds(16, 16)] = x
            x = x + bb_v[pl.ds(12, 16)]
            bb_v[pl.ds(16, 16)] = x
            x = x + bb_v[pl.ds(8, 16)]
            ranks = cnt + x - 1
            jv = c * 16 + io
            rk_v[pl.ds(c * 16, 16)] = jnp.where(m, ranks, _QTRASH)
            return cnt + x[15]

        cnt = lax.fori_loop(0, _NQ // 16, body, jnp.int32(0))

        def body2(c, carry):
            r = c * 16 + io
            m2c = mask_v[pl.ds(c * 16, 16)]
            off3 = jnp.minimum(c * 16 + _N, _NQ - 16)
            m3c = mask_v[pl.ds(off3, 16)]
            rk2 = rk_v[pl.ds(c * 16, 16)]
            rk3 = rk_v[pl.ds(off3, 16)]
            hsrc_v[pl.ds(c * 16, 16)] = jnp.where(
                r < _N,
                jnp.where(m3c != 0, rk3,
                          jnp.where(m2c != 0, rk2, _CBASE + r)),
                _CBASE + r)
            return carry

        lax.fori_loop(0, _NH // 16, body2, 0)
        cnt_v[...] = jnp.zeros((16,), jnp.int32) + cnt
        pltpu.sync_copy(rk_v, rk_hbm)
        pltpu.sync_copy(hsrc_v, hsrc_hbm)
        pltpu.sync_copy(cnt_v, cnt_hbm)


def _qscatter_kernel(no_hbm, un_hbm, rk_hbm, qc_hbm, idx_v, rows_v, sem):
    w = lax.axis_index("s") * 2 + lax.axis_index("c")
    base = w * _RB
    pltpu.sync_copy(rk_hbm.at[pl.ds(base, _RB)], idx_v)

    nsplit = _N // _RB
    @pl.when(w < nsplit)
    def _():
        pltpu.sync_copy(no_hbm.at[pl.ds(base, _RB)], rows_v)

    @pl.when(w == nsplit)
    def _():
        lo = _N - nsplit * _RB
        pltpu.sync_copy(no_hbm.at[pl.ds(nsplit * _RB, lo)],
                        rows_v.at[pl.ds(0, lo)])
        pltpu.sync_copy(un_hbm.at[pl.ds(0, _RB - lo)],
                        rows_v.at[pl.ds(lo, _RB - lo)])

    @pl.when((w > nsplit) & (base - _N + _RB <= _N))
    def _():
        pltpu.sync_copy(un_hbm.at[pl.ds(base - _N, _RB)], rows_v)

    @pl.when(base - _N + _RB > _N)
    def _():
        pltpu.sync_copy(un_hbm.at[pl.ds(base - _N, 2 * _N - base)],
                        rows_v.at[pl.ds(0, 2 * _N - base)])

    def remap(c, carry):
        v = idx_v[pl.ds(c * 16, 16)]
        idx_v[pl.ds(c * 16, 16)] = jnp.where(v == _QTRASH, _QTRASH + w, v)
        return carry

    lax.fori_loop(0, _RB // 16, remap, 0)
    pltpu.async_copy(rows_v, qc_hbm.at[idx_v], sem).wait()


def _att_kernel(cnt_sref, qc_ref, kn_ref, ku_ref, vh_ref, bn_ref, bu_ref,
                m1max_ref, wq_ref, bq_ref,
                att_ref, m23max_ref, out_ref):
    i = pl.program_id(0)
    n23 = cnt_sref[0]
    inv = jnp.float32(1.0) / jnp.sqrt(jnp.float32(_A))

    @pl.when(i == 0)
    def _():
        m23max_ref[...] = jnp.full_like(m23max_ref[...], -jnp.inf)

    @pl.when(i * _RB < n23)
    def _():
        qsrc = qc_ref[...]
        qb = (_mm_nt(qsrc, wq_ref[...]) + bq_ref[...]).astype(jnp.bfloat16)
        p = (jnp.exp(_mm_nt(qb, kn_ref[...]) * inv + bn_ref[...]) +
             jnp.exp(_mm_nt(qb, ku_ref[...]) * inv + bu_ref[...]))
        l = jnp.sum(p, axis=1, keepdims=True)
        att = jnp.dot(p.astype(jnp.bfloat16), vh_ref[...],
                      preferred_element_type=jnp.float32) / l
        ath = jnp.tanh(att)
        att_ref[...] = ath

        rows = i * _RB + jax.lax.broadcasted_iota(jnp.int32, (_RB, 1), 0)
        bm = jnp.max(jnp.where(rows < n23, ath, -jnp.inf),
                     axis=0, keepdims=True)
        m23max_ref[...] = jnp.maximum(m23max_ref[...], bm)

    @pl.when(i == pl.num_programs(0) - 1)
    def _():
        out_m1 = m1max_ref[...]
        out_m23 = m23max_ref[...]
        has1 = jnp.isfinite(jnp.max(out_m1))
        has23 = jnp.isfinite(jnp.max(out_m23))
        out_ref[...] = jnp.where(
            ~has1, out_m23,
            jnp.where(~has23, out_m1, jnp.maximum(out_m1, out_m23)))


def _hmerge_kernel(cat_hbm, hsrc_hbm, hnew_hbm, sidx_v, crows_v, sem):
    w = lax.axis_index("s") * 2 + lax.axis_index("c")
    base = w * _CPR
    pltpu.sync_copy(hsrc_hbm.at[pl.ds(base, _CPR)], sidx_v)
    pltpu.async_copy(cat_hbm.at[sidx_v], crows_v, sem).wait()

    @pl.when(base + _CPR <= _N)
    def _():
        pltpu.sync_copy(crows_v, hnew_hbm.at[pl.ds(base, _CPR)])

    @pl.when(base + _CPR > _N)
    def _():
        pltpu.sync_copy(crows_v.at[pl.ds(0, _N - (31 * _CPR))],
                        hnew_hbm.at[pl.ds(31 * _CPR, _N - (31 * _CPR))])


def kernel(t, co_embeddings, divided, no_embeddings, unrelated_embeddings,
           hidden_state, w_ih, w_hh, b_ih, b_hh, wq, bq, wk, bk, wv, bv):
    f32 = jnp.float32
    bf16 = jnp.bfloat16
    i32 = jnp.int32
    h = (hidden_state if hidden_state is not None
         else jnp.zeros((_N, _D), dtype=co_embeddings.dtype))
    tpos = jnp.asarray(t) > 0
    dv_eff = jnp.where(tpos, divided, divided * jnp.array([1, 0, 0], divided.dtype))

    nblk1 = _N // _R1
    row_spec1 = lambda w: pl.BlockSpec((_R1, w), lambda i: (i, 0))
    full = lambda a: pl.BlockSpec(a.shape, lambda i, *_: tuple(0 for _ in a.shape))

    bih2 = b_ih.reshape(1, -1)
    bhh2 = b_hh.reshape(1, -1)
    bq2 = bq.reshape(1, -1)
    bk2 = bk.reshape(1, -1)
    bv2 = bv.reshape(1, -1)

    hnew0, k_n, k_u, v_h, m1max, kmax2, qmax2 = pl.pallas_call(
        _prep_kernel,
        grid=(nblk1,),
        in_specs=[row_spec1(_D), row_spec1(_D), row_spec1(_D), row_spec1(_D),
                  row_spec1(3),
                  full(w_ih), full(w_hh), full(bih2), full(bhh2),
                  full(wk), full(bk2), full(wv), full(bv2),
                  full(wq), full(bq2)],
        out_specs=[row_spec1(_D), row_spec1(_A), row_spec1(_A), row_spec1(_D),
                   pl.BlockSpec((1, _D), lambda i: (0, 0)),
                   pl.BlockSpec((1, _D), lambda i: (0, 0)),
                   pl.BlockSpec((1, _D), lambda i: (0, 0))],
        out_shape=[
            jax.ShapeDtypeStruct((_NH, _D), f32),
            jax.ShapeDtypeStruct((_N, _A), bf16),
            jax.ShapeDtypeStruct((_N, _A), bf16),
            jax.ShapeDtypeStruct((_N, _D), bf16),
            jax.ShapeDtypeStruct((1, _D), f32),
            jax.ShapeDtypeStruct((1, _D), f32),
            jax.ShapeDtypeStruct((1, _D), f32),
        ],
    )(co_embeddings, no_embeddings, unrelated_embeddings, h, dv_eff,
      w_ih, w_hh, bih2, bhh2, wk, bk2, wv, bv2, wq, bq2)

    mask2 = dv_eff[:, 1].astype(i32)
    mask3 = dv_eff[:, 2].astype(i32)
    m23pad = jnp.concatenate([mask2, mask3, jnp.zeros((_NQ - 2 * _N,), i32)])
    m_g = jnp.sqrt(qmax2[0, 0] * kmax2[0, 0]) * (1.0 / jnp.sqrt(f32(_A))) + f32(1e-3)
    bias_n = jnp.where(mask2 > 0, f32(0), f32(_NEGBIAS)).reshape(1, _N) - m_g
    bias_u = jnp.where(mask3 > 0, f32(0), f32(_NEGBIAS)).reshape(1, _N) - m_g

    mesh = plsc.VectorSubcoreMesh(core_axis_name="c", subcore_axis_name="s")

    rk, hsrc, cnt = pl.kernel(
        _rank_kernel,
        out_type=[
            jax.ShapeDtypeStruct((_NQ,), i32),
            jax.ShapeDtypeStruct((_NH,), i32),
            jax.ShapeDtypeStruct((16,), i32),
        ],
        mesh=mesh,
        scratch_types=[
            pltpu.VMEM((_NQ,), i32),
            pltpu.VMEM((_NQ,), i32),
            pltpu.VMEM((_NH,), i32),
            pltpu.VMEM((32,), i32),
            pltpu.VMEM((16,), i32),
            pltpu.SemaphoreType.DMA,
        ],
    )(m23pad)

    q_c = pl.kernel(
        _qscatter_kernel,
        out_type=jax.ShapeDtypeStruct((_NQ + 32, _D), f32),
        mesh=mesh,
        scratch_types=[
            pltpu.VMEM((_RB,), i32),
            pltpu.VMEM((_RB, _D), f32),
            pltpu.SemaphoreType.DMA,
        ],
    )(no_embeddings, unrelated_embeddings, rk)

    att_c, m23max, outv = pl.pallas_call(
        _att_kernel,
        grid_spec=pltpu.PrefetchScalarGridSpec(
            num_scalar_prefetch=1,
            grid=(_NQ // _RB,),
            in_specs=[pl.BlockSpec((_RB, _D), lambda i, *_: (i, 0)),
                      full(k_n), full(k_u), full(v_h),
                      full(bias_n), full(bias_u),
                      full(m1max), full(wq), full(bq2)],
            out_specs=[pl.BlockSpec((_RB, _D), lambda i, *_: (i, 0)),
                       pl.BlockSpec((1, _D), lambda i, *_: (0, 0)),
                       pl.BlockSpec((1, _D), lambda i, *_: (0, 0))],
        ),
        out_shape=[
            jax.ShapeDtypeStruct((_NQ + 32, _D), f32),
            jax.ShapeDtypeStruct((1, _D), f32),
            jax.ShapeDtypeStruct((1, _D), f32),
        ],
    )(cnt, q_c, k_n, k_u, v_h, bias_n, bias_u, m1max, wq, bq2)

    cat = jnp.concatenate([att_c, hnew0], axis=0)
    h_new = pl.kernel(
        _hmerge_kernel,
        out_type=jax.ShapeDtypeStruct((_N, _D), f32),
        mesh=mesh,
        scratch_types=[
            pltpu.VMEM((_CPR,), i32),
            pltpu.VMEM((_CPR, _D), f32),
            pltpu.SemaphoreType.DMA,
        ],
    )(cat, hsrc)

    return (outv[0], h_new)
